# baseline (speedup 1.0000x reference)
import functools

import jax
import jax.numpy as jnp
from jax import lax
from jax.experimental import pallas as pl
from jax.experimental.pallas import tpu as pltpu
from jax.experimental.pallas import tpu_sc as plsc

VOCAB_SIZE = 49408
HIDDEN_SIZE = 512
MAX_POS = 77
BATCH = 4096
SEQ = 77

N = BATCH * SEQ
NC = 2
NS = 16
NW = NC * NS
PER_W = N // NW
C = 64
NCHUNK = PER_W // C
NBUF = 2
LANES = 16
NGROUP = C // LANES

assert PER_W * NW == N and NCHUNK * C == PER_W and NCHUNK % NBUF == 0

_mesh = plsc.VectorSubcoreMesh(core_axis_name="c", subcore_axis_name="s")


@functools.partial(
    pl.kernel,
    out_type=jax.ShapeDtypeStruct((N, HIDDEN_SIZE), jnp.float32),
    mesh=_mesh,
    compiler_params=pltpu.CompilerParams(
        use_tc_tiling_on_sc=False, needs_layout_passes=False),
    scratch_types=[
        pltpu.VMEM((PER_W,), jnp.int32),
        pltpu.VMEM((PER_W,), jnp.int32),
        pltpu.VMEM((MAX_POS * HIDDEN_SIZE,), jnp.float32),
        pltpu.VMEM((C, HIDDEN_SIZE), jnp.float32),
        pltpu.VMEM((C, HIDDEN_SIZE), jnp.float32),
        pltpu.SemaphoreType.DMA,
        pltpu.SemaphoreType.DMA,
        pltpu.SemaphoreType.DMA,
        pltpu.SemaphoreType.DMA,
    ],
)
def _emb_lookup(ids_hbm, pids_hbm, tok_hbm, pos_hbm, out_hbm,
                idx_t, idx_p, pos_v, buf0, buf1,
                semt0, semt1, semo0, semo1):
    wid = lax.axis_index("s") * NC + lax.axis_index("c")
    w_base = wid * PER_W

    bufs = (buf0, buf1)
    semt = (semt0, semt1)
    semo = (semo0, semo1)

    pltpu.sync_copy(ids_hbm.at[pl.ds(w_base, PER_W)], idx_t)
    pltpu.sync_copy(pids_hbm.at[pl.ds(w_base, PER_W)], idx_p)
    pltpu.sync_copy(pos_hbm, pos_v)

    def tok_copy(g, b):
        return pltpu.make_async_copy(
            tok_hbm.at[idx_t.at[pl.ds(g * C, C)]],
            bufs[b], semt[b])

    def out_copy(g, b):
        return pltpu.make_async_copy(
            bufs[b],
            out_hbm.at[pl.ds(w_base + g * C, C)],
            semo[b])

    tok_copy(0, 0).start()

    def superstep(kk, carry):
        for b in range(NBUF):
            g = NBUF * kk + b
            ob = 1 - b
            tok_copy(g, b).wait()

            @pl.when(g >= 1)
            def _():
                out_copy(g - 1, ob).wait()

            @pl.when(g + 1 < NCHUNK)
            def _():
                tok_copy(g + 1, ob).start()

            @plsc.parallel_loop(0, NGROUP, 1, unroll=2)
            def add_group(k):
                p_vec = idx_p[pl.ds(g * C + k * LANES, LANES)]
                for r16 in range(LANES):
                    p_r = p_vec[r16]
                    r = k * LANES + r16
                    row_out = bufs[b].at[r]
                    row_pos = pos_v.at[pl.ds(p_r * HIDDEN_SIZE, HIDDEN_SIZE)]
                    for j in range(HIDDEN_SIZE // LANES):
                        sl = pl.ds(j * LANES, LANES)
                        plsc.addupdate(row_out.at[sl], row_pos[sl])
            out_copy(g, b).start()
        return carry

    lax.fori_loop(0, NCHUNK // NBUF, superstep, 0)
    out_copy(NCHUNK - 1, (NCHUNK - 1) % NBUF).wait()


def kernel(input_ids, position_ids, token_embedding, position_embedding):
    ids = input_ids.reshape(N).astype(jnp.int32)
    pids = position_ids.reshape(N).astype(jnp.int32)
    pos_flat = position_embedding.reshape(MAX_POS * HIDDEN_SIZE)
    out = _emb_lookup(ids, pids, token_embedding, pos_flat)
    return out.reshape(BATCH, SEQ, HIDDEN_SIZE)


# speedup vs baseline: 1.0260x; 1.0260x over previous
import functools

import jax
import jax.numpy as jnp
from jax import lax
from jax.experimental import pallas as pl
from jax.experimental.pallas import tpu as pltpu
from jax.experimental.pallas import tpu_sc as plsc

VOCAB_SIZE = 49408
HIDDEN_SIZE = 512
MAX_POS = 77
BATCH = 4096
SEQ = 77

N = BATCH * SEQ
NC = 2
NS = 16
NW = NC * NS
PER_W = N // NW
C = 64
NCHUNK = PER_W // C
NBUF = 2
LANES = 16
NGROUP = C // LANES

assert PER_W * NW == N and NCHUNK * C == PER_W and NCHUNK % NBUF == 0

_mesh = plsc.VectorSubcoreMesh(core_axis_name="c", subcore_axis_name="s")


@functools.partial(
    pl.kernel,
    out_type=jax.ShapeDtypeStruct((N, HIDDEN_SIZE), jnp.float32),
    mesh=_mesh,
    scratch_types=[
        pltpu.VMEM((PER_W,), jnp.int32),
        pltpu.VMEM((PER_W,), jnp.int32),
        pltpu.VMEM((MAX_POS * HIDDEN_SIZE,), jnp.float32),
        pltpu.VMEM((C, HIDDEN_SIZE), jnp.float32),
        pltpu.VMEM((C, HIDDEN_SIZE), jnp.float32),
        pltpu.SemaphoreType.DMA,
        pltpu.SemaphoreType.DMA,
        pltpu.SemaphoreType.DMA,
        pltpu.SemaphoreType.DMA,
    ],
)
def _emb_lookup(ids_hbm, pids_hbm, tok_hbm, pos_hbm, out_hbm,
                idx_t, idx_p, pos_v, buf0, buf1,
                semt0, semt1, semo0, semo1):
    wid = lax.axis_index("s") * NC + lax.axis_index("c")
    w_base = wid * PER_W

    bufs = (buf0, buf1)
    semt = (semt0, semt1)
    semo = (semo0, semo1)

    pltpu.sync_copy(ids_hbm.at[pl.ds(w_base, PER_W)], idx_t)
    pltpu.sync_copy(pids_hbm.at[pl.ds(w_base, PER_W)], idx_p)
    pltpu.sync_copy(pos_hbm, pos_v)

    def tok_copy(g, b):
        return pltpu.make_async_copy(
            tok_hbm.at[idx_t.at[pl.ds(g * C, C)]],
            bufs[b], semt[b])

    def out_copy(g, b):
        return pltpu.make_async_copy(
            bufs[b],
            out_hbm.at[pl.ds(w_base + g * C, C)],
            semo[b])

    tok_copy(0, 0).start()

    def superstep(kk, carry):
        for b in range(NBUF):
            g = NBUF * kk + b
            ob = 1 - b
            tok_copy(g, b).wait()

            @pl.when(g >= 1)
            def _():
                out_copy(g - 1, ob).wait()

            @pl.when(g + 1 < NCHUNK)
            def _():
                tok_copy(g + 1, ob).start()

            @plsc.parallel_loop(0, NGROUP, 1, unroll=2)
            def add_group(k):
                p_vec = idx_p[pl.ds(g * C + k * LANES, LANES)]
                for r16 in range(LANES):
                    p_r = p_vec[r16]
                    r = k * LANES + r16
                    row_out = bufs[b].at[r]
                    row_pos = pos_v.at[pl.ds(p_r * HIDDEN_SIZE, HIDDEN_SIZE)]
                    for j in range(HIDDEN_SIZE // LANES):
                        sl = pl.ds(j * LANES, LANES)
                        plsc.addupdate(row_out.at[sl], row_pos[sl])
            out_copy(g, b).start()
        return carry

    lax.fori_loop(0, NCHUNK // NBUF, superstep, 0)
    out_copy(NCHUNK - 1, (NCHUNK - 1) % NBUF).wait()


def kernel(input_ids, position_ids, token_embedding, position_embedding):
    ids = input_ids.reshape(N).astype(jnp.int32)
    pids = position_ids.reshape(N).astype(jnp.int32)
    pos_flat = position_embedding.reshape(MAX_POS * HIDDEN_SIZE)
    out = _emb_lookup(ids, pids, token_embedding, pos_flat)
    return out.reshape(BATCH, SEQ, HIDDEN_SIZE)


# speedup vs baseline: 1.4229x; 1.3869x over previous
import functools

import jax
import jax.numpy as jnp
from jax import lax
from jax.experimental import pallas as pl
from jax.experimental.pallas import tpu as pltpu
from jax.experimental.pallas import tpu_sc as plsc

VOCAB_SIZE = 49408
HIDDEN_SIZE = 512
MAX_POS = 77
BATCH = 4096
SEQ = 77

NC = 2
NS = 16
NW = NC * NS
PER_W = BATCH // NW
NBUF = 2
LANES = 16
SEQ_PAD = 80
IDS_PAD = 128
NGROUP = SEQ_PAD // LANES

assert PER_W * NW == BATCH and PER_W % NBUF == 0

_mesh = plsc.VectorSubcoreMesh(core_axis_name="c", subcore_axis_name="s")


@functools.partial(
    pl.kernel,
    out_type=jax.ShapeDtypeStruct((BATCH, SEQ_PAD, HIDDEN_SIZE), jnp.float32),
    mesh=_mesh,
    scratch_types=[
        pltpu.VMEM((IDS_PAD,), jnp.int32),
        pltpu.VMEM((IDS_PAD,), jnp.int32),
        pltpu.VMEM((IDS_PAD,), jnp.int32),
        pltpu.VMEM((IDS_PAD,), jnp.int32),
        pltpu.VMEM((MAX_POS, HIDDEN_SIZE), jnp.float32),
        pltpu.VMEM((SEQ_PAD, HIDDEN_SIZE), jnp.float32),
        pltpu.VMEM((SEQ_PAD, HIDDEN_SIZE), jnp.float32),
        pltpu.SemaphoreType.DMA,
        pltpu.SemaphoreType.DMA,
        pltpu.SemaphoreType.DMA,
        pltpu.SemaphoreType.DMA,
    ],
)
def _emb_lookup(ids_hbm, pids_hbm, tok_hbm, pos_hbm, out_hbm,
                idxt0, idxt1, idxp0, idxp1, pos_v, buf0, buf1,
                semt0, semt1, semo0, semo1):
    wid = lax.axis_index("s") * NC + lax.axis_index("c")
    w_base = wid * PER_W

    bufs = (buf0, buf1)
    idxt = (idxt0, idxt1)
    idxp = (idxp0, idxp1)
    semt = (semt0, semt1)
    semo = (semo0, semo1)

    pltpu.sync_copy(pos_hbm, pos_v)

    def stage_idx(g, b):
        bi = w_base + g
        pltpu.sync_copy(ids_hbm.at[bi], idxt[b])
        pltpu.sync_copy(pids_hbm.at[bi], idxp[b])

    def tok_copy(g, b):
        return pltpu.make_async_copy(
            tok_hbm.at[idxt[b].at[pl.ds(0, SEQ_PAD)]], bufs[b], semt[b])

    def out_copy(g, b):
        return pltpu.make_async_copy(
            bufs[b], out_hbm.at[w_base + g], semo[b])

    def add_rows(b):
        def add_group(k, carry2):
            base = k * LANES
            p_vec = idxp[b][pl.ds(base, LANES)]
            for r16 in range(LANES):
                p_r = p_vec[r16]
                r = base + r16
                for j in range(HIDDEN_SIZE // LANES):
                    sl = pl.ds(j * LANES, LANES)
                    plsc.addupdate(bufs[b].at[r, sl], pos_v[p_r, sl])
            return carry2

        lax.fori_loop(0, NGROUP, add_group, 0)

    stage_idx(0, 0)
    tok_copy(0, 0).start()

    def superstep(kk, carry):
        for b in range(NBUF):
            g = NBUF * kk + b
            ob = 1 - b
            tok_copy(g, b).wait()

            @pl.when(g >= 1)
            def _():
                out_copy(g - 1, ob).wait()

            @pl.when(g + 1 < PER_W)
            def _():
                stage_idx(g + 1, ob)
                tok_copy(g + 1, ob).start()

            add_rows(b)
            out_copy(g, b).start()
        return carry

    lax.fori_loop(0, PER_W // NBUF, superstep, 0)
    out_copy(PER_W - 1, (PER_W - 1) % NBUF).wait()


def kernel(input_ids, position_ids, token_embedding, position_embedding):
    ids = jnp.pad(input_ids.astype(jnp.int32), ((0, 0), (0, IDS_PAD - SEQ)))
    pids = jnp.pad(position_ids.astype(jnp.int32), ((0, 0), (0, IDS_PAD - SEQ)))
    out = _emb_lookup(ids, pids, token_embedding, position_embedding)
    return out[:, :SEQ, :]


# speedup vs baseline: 1.5037x; 1.0568x over previous
import functools

import jax
import jax.numpy as jnp
from jax import lax
from jax.experimental import pallas as pl
from jax.experimental.pallas import tpu as pltpu
from jax.experimental.pallas import tpu_sc as plsc

VOCAB_SIZE = 49408
HIDDEN_SIZE = 512
MAX_POS = 77
BATCH = 4096
SEQ = 77

NC = 2
NS = 16
NW = NC * NS
PER_W = BATCH // NW
NBUF = 2
LANES = 16
SEQ_PAD = 80
IDS_PAD = 128
NGROUP = SEQ_PAD // LANES

assert PER_W * NW == BATCH and PER_W % NBUF == 0

_mesh = plsc.VectorSubcoreMesh(core_axis_name="c", subcore_axis_name="s")


@functools.partial(
    pl.kernel,
    out_type=jax.ShapeDtypeStruct((BATCH, SEQ_PAD, HIDDEN_SIZE), jnp.float32),
    mesh=_mesh,
    scratch_types=[
        pltpu.VMEM((IDS_PAD,), jnp.int32),
        pltpu.VMEM((IDS_PAD,), jnp.int32),
        pltpu.VMEM((IDS_PAD,), jnp.int32),
        pltpu.VMEM((IDS_PAD,), jnp.int32),
        pltpu.VMEM((MAX_POS, HIDDEN_SIZE), jnp.float32),
        pltpu.VMEM((SEQ_PAD, HIDDEN_SIZE), jnp.float32),
        pltpu.VMEM((SEQ_PAD, HIDDEN_SIZE), jnp.float32),
        pltpu.SemaphoreType.DMA,
        pltpu.SemaphoreType.DMA,
        pltpu.SemaphoreType.DMA,
        pltpu.SemaphoreType.DMA,
        pltpu.SemaphoreType.DMA,
        pltpu.SemaphoreType.DMA,
    ],
)
def _emb_lookup(ids_hbm, pids_hbm, tok_hbm, pos_hbm, out_hbm,
                idxt0, idxt1, idxp0, idxp1, pos_v, buf0, buf1,
                semt0, semt1, semo0, semo1, semi0, semi1):
    wid = lax.axis_index("s") * NC + lax.axis_index("c")
    w_base = wid * PER_W

    bufs = (buf0, buf1)
    idxt = (idxt0, idxt1)
    idxp = (idxp0, idxp1)
    semt = (semt0, semt1)
    semo = (semo0, semo1)
    semi = (semi0, semi1)

    pltpu.sync_copy(pos_hbm, pos_v)

    def stage_t(g, b):
        return pltpu.make_async_copy(ids_hbm.at[w_base + g], idxt[b], semi[b])

    def stage_p(g, b):
        return pltpu.make_async_copy(pids_hbm.at[w_base + g], idxp[b], semi[b])

    def stage_start(g, b):
        stage_t(g, b).start()
        stage_p(g, b).start()

    def stage_wait(g, b):
        stage_t(g, b).wait()
        stage_p(g, b).wait()

    def tok_copy(g, b):
        return pltpu.make_async_copy(
            tok_hbm.at[idxt[b].at[pl.ds(0, SEQ_PAD)]], bufs[b], semt[b])

    def out_copy(g, b):
        return pltpu.make_async_copy(
            bufs[b], out_hbm.at[w_base + g], semo[b])

    def add_rows(b):
        def add_group(k, carry2):
            base = k * LANES
            p_vec = idxp[b][pl.ds(base, LANES)]
            for r16 in range(LANES):
                p_r = p_vec[r16]
                r = base + r16
                for j in range(HIDDEN_SIZE // LANES):
                    sl = pl.ds(j * LANES, LANES)
                    plsc.addupdate(bufs[b].at[r, sl], pos_v[p_r, sl])
            return carry2

        lax.fori_loop(0, NGROUP, add_group, 0)

    stage_start(0, 0)
    stage_wait(0, 0)
    tok_copy(0, 0).start()
    stage_start(1, 1)

    def superstep(kk, carry):
        for b in range(NBUF):
            g = NBUF * kk + b
            ob = 1 - b
            tok_copy(g, b).wait()

            @pl.when(g >= 1)
            def _():
                out_copy(g - 1, ob).wait()

            @pl.when(g + 1 < PER_W)
            def _():
                stage_wait(g + 1, ob)
                tok_copy(g + 1, ob).start()

            add_rows(b)

            @pl.when(g + 2 < PER_W)
            def _():
                stage_start(g + 2, b)

            out_copy(g, b).start()
        return carry

    lax.fori_loop(0, PER_W // NBUF, superstep, 0)
    out_copy(PER_W - 1, (PER_W - 1) % NBUF).wait()


def kernel(input_ids, position_ids, token_embedding, position_embedding):
    ids = jnp.pad(input_ids.astype(jnp.int32), ((0, 0), (0, IDS_PAD - SEQ)))
    pids = jnp.pad(position_ids.astype(jnp.int32), ((0, 0), (0, IDS_PAD - SEQ)))
    out = _emb_lookup(ids, pids, token_embedding, position_embedding)
    return out[:, :SEQ, :]
